# baseline (speedup 1.0000x reference)
import functools

import jax
import jax.numpy as jnp
from jax import lax
from jax.experimental import pallas as pl
from jax.experimental.pallas import tpu as pltpu
from jax.experimental.pallas import tpu_sc as plsc

_NUM_EMB = 1000000
_D = 64
_B = 4096
_H = 50

_INFO = plsc.get_sparse_core_info()
_NC = _INFO.num_cores
_NS = _INFO.num_subcores
_NW = _NC * _NS
_TOTAL = _B * _H
_BPW = _TOTAL // _NW
_CH = 128
_NCH = _BPW // _CH
_NBUF = 5


def _body(idx_hbm, table_hbm, out_hbm, idx_v, *rest):
  bufs = rest[:_NBUF]
  gsems = rest[_NBUF:2 * _NBUF]
  wsems = rest[2 * _NBUF:3 * _NBUF]

  wid = lax.axis_index("s") * _NC + lax.axis_index("c")
  base = wid * _BPW

  pltpu.sync_copy(idx_hbm.at[wid], idx_v)

  def start_gather(chunk, b):
    pltpu.async_copy(table_hbm.at[idx_v.at[chunk]], bufs[b], gsems[b])

  def wait_gather(chunk, b):
    pltpu.make_async_copy(
        table_hbm.at[idx_v.at[chunk]], bufs[b], gsems[b]).wait()

  def start_write(chunk, b):
    pltpu.async_copy(
        bufs[b], out_hbm.at[pl.ds(base + chunk * _CH, _CH)], wsems[b])

  def wait_write(chunk, b):
    pltpu.make_async_copy(
        bufs[b], out_hbm.at[pl.ds(base + chunk * _CH, _CH)], wsems[b]).wait()

  for b in range(_NBUF):
    start_gather(b, b)

  @pl.loop(0, _NCH - _NBUF, step=_NBUF)
  def _(i0):
    for b in range(_NBUF):
      i = i0 + b
      wait_gather(i, b)
      start_write(i, b)
      wait_write(i, b)
      start_gather(i + _NBUF, b)

  for b in range(_NBUF):
    i = _NCH - _NBUF + b
    wait_gather(i, b)
    start_write(i, b)
  for b in range(_NBUF):
    i = _NCH - _NBUF + b
    wait_write(i, b)


@jax.jit
def _run(idx, table):
  mesh = plsc.VectorSubcoreMesh(core_axis_name="c", subcore_axis_name="s")
  scratch = (
      [pltpu.VMEM((_NCH, _CH), jnp.int32)]
      + [pltpu.VMEM((_CH, _D), jnp.float32) for _ in range(_NBUF)]
      + [pltpu.SemaphoreType.DMA for _ in range(2 * _NBUF)]
  )
  out = pl.kernel(
      _body,
      out_type=jax.ShapeDtypeStruct((_TOTAL, _D), jnp.float32),
      mesh=mesh,
      scratch_types=scratch,
      compiler_params=pltpu.CompilerParams(use_tc_tiling_on_sc=False),
  )(idx, table)
  return out


def kernel(indices, table):
  idx = indices.astype(jnp.int32).reshape(_NW, _NCH, _CH)
  out = _run(idx, table)
  return out.reshape(_B, _H, _D)


# speedup vs baseline: 1.0166x; 1.0166x over previous
import functools

import jax
import jax.numpy as jnp
from jax import lax
from jax.experimental import pallas as pl
from jax.experimental.pallas import tpu as pltpu
from jax.experimental.pallas import tpu_sc as plsc

_NUM_EMB = 1000000
_D = 64
_B = 4096
_H = 50

_INFO = plsc.get_sparse_core_info()
_NC = _INFO.num_cores
_NS = _INFO.num_subcores
_NW = _NC * _NS
_TOTAL = _B * _H
_BPW = _TOTAL // _NW
_CH = 128
_NCH = _BPW // _CH
_NBUF = 5


def _body(idx_hbm, table_hbm, out_hbm, idx_v, *rest):
  bufs = rest[:_NBUF]
  gsems = rest[_NBUF:2 * _NBUF]
  wsems = rest[2 * _NBUF:3 * _NBUF]

  wid = lax.axis_index("s") * _NC + lax.axis_index("c")
  base = wid * _BPW

  pltpu.sync_copy(idx_hbm.at[wid], idx_v)

  def start_gather(chunk, b):
    pltpu.async_copy(table_hbm.at[idx_v.at[chunk]], bufs[b], gsems[b])

  def wait_gather(chunk, b):
    pltpu.make_async_copy(
        table_hbm.at[idx_v.at[chunk]], bufs[b], gsems[b]).wait()

  def start_write(chunk, b):
    pltpu.async_copy(
        bufs[b], out_hbm.at[pl.ds(base + chunk * _CH, _CH)], wsems[b])

  def wait_write(chunk, b):
    pltpu.make_async_copy(
        bufs[b], out_hbm.at[pl.ds(base + chunk * _CH, _CH)], wsems[b]).wait()

  for b in range(_NBUF):
    start_gather(b, b)

  @pl.loop(0, _NCH - _NBUF, step=_NBUF)
  def _(i0):
    for b in range(_NBUF):
      i = i0 + b
      wait_gather(i, b)
      start_write(i, b)
      wait_write(i, b)
      start_gather(i + _NBUF, b)

  for b in range(_NBUF):
    i = _NCH - _NBUF + b
    wait_gather(i, b)
    start_write(i, b)
  for b in range(_NBUF):
    i = _NCH - _NBUF + b
    wait_write(i, b)


@jax.jit
def _run(idx, table):
  mesh = plsc.VectorSubcoreMesh(core_axis_name="c", subcore_axis_name="s")
  scratch = (
      [pltpu.VMEM((_NCH, _CH), jnp.int32)]
      + [pltpu.VMEM((_CH, _D), jnp.float32) for _ in range(_NBUF)]
      + [pltpu.SemaphoreType.DMA for _ in range(2 * _NBUF)]
  )
  out = pl.kernel(
      _body,
      out_type=jax.ShapeDtypeStruct((_TOTAL, _D), jnp.float32),
      mesh=mesh,
      scratch_types=scratch,
      compiler_params=pltpu.CompilerParams(use_tc_tiling_on_sc=False),
  )(idx, table)
  return out


def kernel(indices, table):
  idx = indices.T.astype(jnp.int32).reshape(_NW, _NCH, _CH)
  out = _run(idx, table)
  return out.reshape(_H, _B, _D).transpose(1, 0, 2)
